# baseline (speedup 1.0000x reference)
import functools

import jax
import jax.numpy as jnp
from jax import lax
from jax.experimental import pallas as pl
from jax.experimental.pallas import tpu as pltpu
from jax.experimental.pallas import tpu_sc as plsc

NC = 2
NS = 16
NW = NC * NS
CHUNK = 128


def _mesh():
    return plsc.VectorSubcoreMesh(core_axis_name="c", subcore_axis_name="s")


_SC_PARAMS = pltpu.CompilerParams(use_tc_tiling_on_sc=False)


def _sc_gather2(npad, d, epad, gsz, count=False, name="sc_gather"):
    per_w = epad // NW
    rr = 40 * CHUNK
    nb0 = (2 * per_w // rr) * 4 // 10
    nb1 = 2 * per_w // rr - nb0
    pw0, pw1 = nb0 * rr, nb1 * rr
    ops = rr // gsz
    rpt = npad // NS

    out_t = [jax.ShapeDtypeStruct((epad, d), jnp.float32),
             jax.ShapeDtypeStruct((epad, d), jnp.float32)]
    scratch = [
        pltpu.VMEM((rr,), jnp.int32),
        pltpu.VMEM((rr,), jnp.int32),
        pltpu.VMEM((2, gsz, d), jnp.float32),
        pltpu.VMEM((2, gsz, d), jnp.float32),
        [pltpu.SemaphoreType.DMA] * 2,
        [pltpu.SemaphoreType.DMA] * 2,
    ]
    if count:
        out_t.append(jax.ShapeDtypeStruct((NC * npad, 16), jnp.float32))
        scratch += [
            pltpu.VMEM((gsz, 16), jnp.float32),
            pltpu.VMEM_SHARED((npad, 16), jnp.float32),
            pltpu.SemaphoreType.DMA,
        ]

    @functools.partial(pl.kernel, out_type=tuple(out_t), mesh=_mesh(),
                       compiler_params=_SC_PARAMS, name=name,
                       scratch_types=scratch)
    def k(table, src1, dst1, zeros16, ones, *refs):
        if count:
            (hs, hd, cnt, src_v, dst_v, rs_v, rd_v, gsem, wsem,
             ones_v, acc, csem) = refs
        else:
            hs, hd, src_v, dst_v, rs_v, rd_v, gsem, wsem = refs
        c = lax.axis_index("c")
        s = lax.axis_index("s")
        base = c * NS * pw0 + s * (pw0 + c * (pw1 - pw0))
        nbc = nb0 + c * (nb1 - nb0)
        if count:
            pltpu.sync_copy(ones, ones_v)
            pltpu.sync_copy(zeros16.at[pl.ds(s * rpt, rpt)],
                            acc.at[pl.ds(s * rpt, rpt)])
            plsc.subcore_barrier()

        def fire_gather(o, slot):
            pltpu.async_copy(table.at[src_v.at[pl.ds(o * gsz, gsz)]],
                             rs_v.at[slot], gsem[slot])
            pltpu.async_copy(table.at[dst_v.at[pl.ds(o * gsz, gsz)]],
                             rd_v.at[slot], gsem[slot])

        def drain(ref, buf, sem):
            pltpu.make_async_copy(ref.at[pl.ds(0, gsz)], buf, sem).wait()

        def outer(ob, carry):
            pltpu.sync_copy(dst1.at[pl.ds(base + ob * rr, rr)], dst_v)
            pltpu.sync_copy(src1.at[pl.ds(base + ob * rr, rr)], src_v)
            fire_gather(0, 0)

            def body(ip, carry2):
                for b in range(2):
                    o = ip * 2 + b
                    row = base + ob * rr + o * gsz
                    drain(hs, rs_v.at[b], gsem[b])
                    drain(hd, rd_v.at[b], gsem[b])
                    pltpu.async_copy(rs_v.at[b], hs.at[pl.ds(row, gsz)],
                                     wsem[b])
                    pltpu.async_copy(rd_v.at[b], hd.at[pl.ds(row, gsz)],
                                     wsem[b])
                    if count:
                        pltpu.async_copy(
                            ones_v, acc.at[dst_v.at[pl.ds(o * gsz, gsz)]],
                            csem, add=True)

                        @pl.when(ob * ops + o >= 2)
                        def _():
                            pltpu.make_async_copy(ones, ones_v,
                                                  csem).wait()
                    b1 = (b + 1) % 2

                    @pl.when(o + 1 < ops)
                    def _():
                        @pl.when(o >= 1)
                        def _():
                            drain(hs, rs_v.at[b1], wsem[b1])
                            drain(hd, rd_v.at[b1], wsem[b1])
                        fire_gather(o + 1, b1)
                return carry2

            lax.fori_loop(0, ops // 2, body, None)
            for b in range(2):
                drain(hs, rs_v.at[b], wsem[b])
                drain(hd, rd_v.at[b], wsem[b])
            return carry

        lax.fori_loop(0, nbc, outer, None)
        if count:
            pltpu.make_async_copy(ones, ones_v, csem).wait()
            pltpu.make_async_copy(ones, ones_v, csem).wait()
            plsc.subcore_barrier()
            pltpu.sync_copy(acc.at[pl.ds(s * rpt, rpt)],
                            cnt.at[pl.ds(c * npad + s * rpt, rpt)])

    return k


def _sc_scatter_add(npad, width, epad, gather_table=False,
                    name="sc_scatter"):
    per_w = epad // NW
    gsz = 2 * CHUNK
    rr = 20 * CHUNK
    nb0 = (2 * per_w // rr) * 4 // 10
    nb1 = 2 * per_w // rr - nb0
    pw0, pw1 = nb0 * rr, nb1 * rr
    ops = rr // gsz
    rpt = npad // NS

    out_t = jax.ShapeDtypeStruct((NC * npad, width), jnp.float32)
    scratch = [
        pltpu.VMEM((rr,), jnp.int32),
        pltpu.VMEM((rr,), jnp.int32),
        pltpu.VMEM((2, gsz, width), jnp.float32),
        pltpu.VMEM_SHARED((npad, width), jnp.float32),
        [pltpu.SemaphoreType.DMA] * 2,
        [pltpu.SemaphoreType.DMA] * 2,
    ]

    @functools.partial(pl.kernel, out_type=out_t, mesh=_mesh(),
                       compiler_params=_SC_PARAMS, name=name,
                       scratch_types=scratch)
    def k(src_data, dst1, zeros, src1, out, dst_v, src_v, buf, acc,
          lsem, ssem):
        c = lax.axis_index("c")
        s = lax.axis_index("s")
        pltpu.sync_copy(zeros.at[pl.ds(s * rpt, rpt)],
                        acc.at[pl.ds(s * rpt, rpt)])
        plsc.subcore_barrier()
        base = c * NS * pw0 + s * (pw0 + c * (pw1 - pw0))
        nbc = nb0 + c * (nb1 - nb0)

        def start(o, ob, b):
            if gather_table:
                pltpu.async_copy(
                    src_data.at[src_v.at[pl.ds(o * gsz, gsz)]],
                    buf.at[b], lsem[b])
            else:
                pltpu.async_copy(
                    src_data.at[pl.ds(base + ob * rr + o * gsz, gsz)],
                    buf.at[b], lsem[b])

        def outer(ob, carry):
            pltpu.sync_copy(dst1.at[pl.ds(base + ob * rr, rr)], dst_v)
            if gather_table:
                pltpu.sync_copy(src1.at[pl.ds(base + ob * rr, rr)], src_v)
            start(0, ob, 0)

            def body(ip, carry2):
                for b in range(2):
                    o = ip * 2 + b
                    b1 = (b + 1) % 2
                    pltpu.make_async_copy(src_data.at[pl.ds(0, gsz)],
                                          buf.at[b], lsem[b]).wait()
                    pltpu.async_copy(buf.at[b],
                                     acc.at[dst_v.at[pl.ds(o * gsz, gsz)]],
                                     ssem[b], add=True)

                    @pl.when(o + 1 < ops)
                    def _():
                        @pl.when(o >= 1)
                        def _():
                            pltpu.make_async_copy(
                                src_data.at[pl.ds(0, gsz)],
                                buf.at[b1], ssem[b1]).wait()
                        start(o + 1, ob, b1)
                return carry2

            lax.fori_loop(0, ops // 2, body, None)
            for b in range(2):
                pltpu.make_async_copy(src_data.at[pl.ds(0, gsz)],
                                      buf.at[b], ssem[b]).wait()
            return carry

        lax.fori_loop(0, nbc, outer, None)
        plsc.subcore_barrier()
        pltpu.sync_copy(acc.at[pl.ds(s * rpt, rpt)],
                        out.at[pl.ds(c * npad + s * rpt, rpt)])

    return k


def _full(shape):
    return pl.BlockSpec(shape, lambda i: tuple(0 for _ in shape))


def _tc_edge_mlp0(epad, be):
    grid = epad // be
    be4 = be // 4

    def body(hs, hd, ef, a, b, cc, b1, w2, b2, out):
        res = []
        for kk in range(4):
            sl = slice(32 * kk, 32 * kk + 32)
            z = jnp.dot(hs[:, sl], a[...],
                        preferred_element_type=jnp.float32)
            z += jnp.dot(hd[:, sl], b[...],
                         preferred_element_type=jnp.float32)
            z += lax.dot_general(ef[kk], cc[...], (((0,), (0,)), ((), ())),
                                 preferred_element_type=jnp.float32)
            z = jnp.maximum(z + b1[...], 0.0)
            res.append(jnp.dot(z, w2[...],
                               preferred_element_type=jnp.float32)
                       + b2[...])
        out[...] = jnp.concatenate(res, axis=1)

    def make(a, b, cc, b1, w2, b2):
        call = pl.pallas_call(
            body, grid=(grid,), name="tc_edge_mlp0",
            in_specs=[
                pl.BlockSpec((be4, 128), lambda i: (i, 0)),
                pl.BlockSpec((be4, 128), lambda i: (i, 0)),
                pl.BlockSpec((4, 4, be4), lambda i: (0, 0, i)),
                _full(a.shape), _full(b.shape), _full(cc.shape),
                _full(b1.shape), _full(w2.shape), _full(b2.shape),
            ],
            out_specs=pl.BlockSpec((be4, 128), lambda i: (i, 0)),
            out_shape=jax.ShapeDtypeStruct((epad // 4, 128), jnp.float32),
        )
        return lambda hs, hd, ef: call(hs, hd, ef, a, b, cc, b1, w2, b2)

    return make


def _tc_edge_mlp1(epad, be):
    grid = epad // be
    be4 = be // 4

    def body(hs1, hd1, hs0, hd0, ef, ah, ax, bh, bx, cc, b1, w2, b2, out):
        res = []
        for kk in range(4):
            sl = slice(32 * kk, 32 * kk + 32)
            z = jnp.dot(hs1[:, sl], ah[...],
                        preferred_element_type=jnp.float32)
            z += jnp.dot(hs0[:, sl], ax[...],
                         preferred_element_type=jnp.float32)
            z += jnp.dot(hd1[:, sl], bh[...],
                         preferred_element_type=jnp.float32)
            z += jnp.dot(hd0[:, sl], bx[...],
                         preferred_element_type=jnp.float32)
            z += jnp.dot(ef[:, sl], cc[...],
                         preferred_element_type=jnp.float32)
            z = jnp.maximum(z + b1[...], 0.0)
            res.append(jnp.dot(z, w2[...],
                               preferred_element_type=jnp.float32)
                       + b2[...])
        out[...] = jnp.concatenate(res, axis=1)

    def make(ah, ax, bh, bx, cc, b1, w2, b2):
        call = pl.pallas_call(
            body, grid=(grid,), name="tc_edge_mlp1",
            in_specs=[
                pl.BlockSpec((be4, 128), lambda i: (i, 0)),
                pl.BlockSpec((be4, 128), lambda i: (i, 0)),
                pl.BlockSpec((be4, 128), lambda i: (i, 0)),
                pl.BlockSpec((be4, 128), lambda i: (i, 0)),
                pl.BlockSpec((be4, 128), lambda i: (i, 0)),
                _full(ah.shape), _full(ax.shape), _full(bh.shape),
                _full(bx.shape), _full(cc.shape), _full(b1.shape),
                _full(w2.shape), _full(b2.shape),
            ],
            out_specs=pl.BlockSpec((be4, 128), lambda i: (i, 0)),
            out_shape=jax.ShapeDtypeStruct((epad // 4, 128), jnp.float32),
        )
        return lambda hs1, hd1, hs0, hd0, ef: call(
            hs1, hd1, hs0, hd0, ef, ah, ax, bh, bx, cc, b1, w2, b2)

    return make


def _tc_node_mlp(npad, bn, extra=False):
    grid = npad // bn
    bn4 = bn // 4

    def body(*args):
        if extra:
            h, hx, aggp, d1, dx, d2, b1, w2, b2, out = args
        else:
            h, aggp, d1, d2, b1, w2, b2, out = args
        agg = aggp[0] + aggp[1]
        res = []
        for kk in range(4):
            sl = slice(32 * kk, 32 * kk + 32)
            z = jnp.dot(h[:, sl], d1[...],
                        preferred_element_type=jnp.float32)
            if extra:
                z += jnp.dot(hx[:, sl], dx[...],
                             preferred_element_type=jnp.float32)
            z += jnp.dot(agg[:, sl], d2[...],
                         preferred_element_type=jnp.float32)
            z = jnp.maximum(z + b1[...], 0.0)
            res.append(jnp.dot(z, w2[...],
                               preferred_element_type=jnp.float32)
                       + b2[...])
        out[...] = jnp.concatenate(res, axis=1)

    def make(d1, dx, d2, b1, w2, b2):
        specs = [pl.BlockSpec((bn4, 128), lambda i: (i, 0))]
        if extra:
            specs.append(pl.BlockSpec((bn4, 128), lambda i: (i, 0)))
        specs.append(pl.BlockSpec((NC, bn4, 128), lambda i: (0, i, 0)))
        specs.append(_full(d1.shape))
        if extra:
            specs.append(_full(dx.shape))
        specs += [_full(d2.shape), _full(b1.shape), _full(w2.shape),
                  _full(b2.shape)]
        call = pl.pallas_call(
            body, grid=(grid,), name="tc_node_mlp",
            in_specs=specs,
            out_specs=pl.BlockSpec((bn4, 128), lambda i: (i, 0)),
            out_shape=jax.ShapeDtypeStruct((npad // 4, 128), jnp.float32),
        )
        if extra:
            return lambda h, hx, aggp: call(h, hx, aggp, d1, dx, d2, b1,
                                            w2, b2)
        return lambda h, aggp: call(h, aggp, d1, d2, b1, w2, b2)

    return make


def _tc_divide(npad, bn):
    grid = npad // bn
    bn4 = bn // 4

    def body(sp, rcp, out):
        out[...] = (sp[0] + sp[1]) * rcp[...]

    return pl.pallas_call(
        body, grid=(grid,), name="tc_divide",
        in_specs=[
            pl.BlockSpec((NC, bn4, 128), lambda i: (0, i, 0)),
            pl.BlockSpec((bn4, 128), lambda i: (i, 0)),
        ],
        out_specs=pl.BlockSpec((bn4, 128), lambda i: (i, 0)),
        out_shape=jax.ShapeDtypeStruct((npad // 4, 128), jnp.float32),
    )


def _tc_final(npad, bn):
    grid = npad // bn
    bn4 = bn // 4

    def body(sp, rcp, h0, wa, wb, bd, out):
        hm = (sp[0] + sp[1]) * rcp[...]
        res = []
        for kk in range(4):
            z = jnp.dot(hm[:, 32 * kk:32 * kk + 32], wa[...],
                        preferred_element_type=jnp.float32)
            z += jnp.dot(h0[:, 32 * kk:32 * kk + 2], wb[...],
                         preferred_element_type=jnp.float32)
            res.append(z + bd[...])
        out[...] = jnp.concatenate(res, axis=1)

    def make(wa, wb, bd):
        call = pl.pallas_call(
            body, grid=(grid,), name="tc_final",
            in_specs=[
                pl.BlockSpec((NC, bn4, 128), lambda i: (0, i, 0)),
                pl.BlockSpec((bn4, 128), lambda i: (i, 0)),
                pl.BlockSpec((bn4, 128), lambda i: (i, 0)),
                _full(wa.shape), _full(wb.shape), _full(bd.shape),
            ],
            out_specs=pl.BlockSpec((bn4, 32), lambda i: (i, 0)),
            out_shape=jax.ShapeDtypeStruct((npad // 4, 32), jnp.float32),
        )
        return lambda sp, rcp, h0: call(sp, rcp, h0, wa, wb, bd)

    return make


def _pad_rows(w, rows):
    return jnp.concatenate(
        [w, jnp.zeros((rows - w.shape[0], w.shape[1]), w.dtype)], axis=0)


def kernel(x, edge_index, edge_attr, params):
    n = x.shape[0]
    e = edge_index.shape[1]
    npad = ((n + 16) + 127) // 128 * 128
    epad = -(-e // (NW * CHUNK * 8)) * (NW * CHUNK * 8)
    bn = npad // 4
    be = 4096

    src = edge_index[0].astype(jnp.int32)
    dst = edge_index[1].astype(jnp.int32)
    src1 = jnp.concatenate([src, jnp.zeros((epad - e,), jnp.int32)])
    dst1 = jnp.concatenate([dst, jnp.full((epad - e,), n, jnp.int32)])
    eat = jnp.concatenate(
        [edge_attr.T.astype(jnp.float32),
         jnp.zeros((4, epad - e), jnp.float32)], axis=1)
    eatp = jnp.stack(
        [lax.slice(eat, (0, k), (4, epad), (1, 4)) for k in range(4)])

    z32 = jnp.zeros((npad, 32), jnp.float32)
    z16 = jnp.zeros((npad, 16), jnp.float32)
    ones16 = jnp.ones((2 * CHUNK, 16), jnp.float32)

    h0p = _pad_rows(jnp.concatenate(
        [x.astype(jnp.float32), jnp.zeros((n, 26), jnp.float32)], axis=1),
        npad)

    p0, p1 = params["proc0"], params["proc1"]
    row = lambda v: v.reshape(1, -1).astype(jnp.float32)
    f32 = lambda v: v.astype(jnp.float32)

    def xmap(wrows):
        m = jnp.zeros((32, 64), jnp.float32)
        m = m.at[0:2].set(wrows[0:2])
        m = m.at[3:6].set(wrows[2:5])
        return m

    a0 = _pad_rows(f32(p0["We1"][0:6]), 32)
    b0 = _pad_rows(f32(p0["We1"][6:12]), 32)
    c0 = f32(p0["We1"][12:16])
    d1_0 = _pad_rows(f32(p0["Wn1"][0:6]), 32)
    d2_0 = f32(p0["Wn1"][6:38])
    a1h, a1x = f32(p1["We1"][0:32]), xmap(f32(p1["We1"][32:37]))
    b1h, b1x = f32(p1["We1"][37:69]), xmap(f32(p1["We1"][69:74]))
    c1 = f32(p1["We1"][74:106])
    d1h, d1x = f32(p1["Wn1"][0:32]), xmap(f32(p1["Wn1"][32:37]))
    d2_1 = f32(p1["Wn1"][37:69])
    wda = jnp.concatenate(
        [f32(params["Wd"][0:32]), jnp.zeros((32, 5), jnp.float32)], axis=1)
    wdb = jnp.concatenate(
        [f32(params["Wd"][32:34]), jnp.zeros((2, 5), jnp.float32)], axis=1)
    bdp = jnp.concatenate(
        [row(params["bd"]), jnp.zeros((1, 5), jnp.float32)], axis=1)

    gather_l0 = _sc_gather2(npad, 32, epad, 2 * CHUNK, count=True,
                            name="sc_gather_l0")
    gather_l1 = _sc_gather2(npad, 32, epad, 4 * CHUNK, name="sc_gather_l1")
    scat_e1 = _sc_scatter_add(npad, 32, epad, name="sc_seg_e1")
    scat_e2 = _sc_scatter_add(npad, 32, epad, name="sc_seg_e2")
    smooth0 = _sc_scatter_add(npad, 32, epad, gather_table=True,
                              name="sc_smooth0")
    smooth1 = _sc_scatter_add(npad, 32, epad, gather_table=True,
                              name="sc_smooth1")
    edge0 = _tc_edge_mlp0(epad, be)(
        a0, b0, c0, row(p0["be1"]), f32(p0["We2"]), row(p0["be2"]))
    edge1 = _tc_edge_mlp1(epad, be)(
        a1h, a1x, b1h, b1x, c1, row(p1["be1"]), f32(p1["We2"]),
        row(p1["be2"]))
    node0 = _tc_node_mlp(npad, bn)(
        d1_0, None, d2_0, row(p0["bn1"]), f32(p0["Wn2"]), row(p0["bn2"]))
    node1 = _tc_node_mlp(npad, bn, extra=True)(
        d1h, d1x, d2_1, row(p1["bn1"]), f32(p1["Wn2"]), row(p1["bn2"]))
    divide = _tc_divide(npad, bn)
    final = _tc_final(npad, bn)(wda, wdb, bdp)

    pk4 = lambda v: v.reshape(v.shape[0] // 4, 128)
    as3p = lambda v: v.reshape(NC, npad // 4, 128)

    hs0, hd0, cnt = gather_l0(h0p, src1, dst1, z16, ones16)
    e1p = edge0(pk4(hs0), pk4(hd0), eatp)
    e1 = e1p.reshape(epad, 32)
    agg0 = scat_e1(e1, dst1, z32, src1)
    h1p = node0(pk4(h0p), as3p(agg0))
    h1 = h1p.reshape(npad, 32)
    s0 = smooth0(h1, dst1, z32, src1)
    cnt2 = cnt.reshape(NC, npad, 16)
    rcp = 1.0 / jnp.maximum(cnt2[0, :, 0:1] + cnt2[1, :, 0:1], 1.0)
    rcp4 = pk4(jnp.broadcast_to(rcp, (npad, 32)).reshape(npad, 32))
    h1mp = divide(as3p(s0), rcp4)
    h1m = h1mp.reshape(npad, 32)
    hs1, hd1 = gather_l1(h1m, src1, dst1, z16, ones16)
    e2p = edge1(pk4(hs1), pk4(hd1), pk4(hs0), pk4(hd0), e1p)
    e2 = e2p.reshape(epad, 32)
    agg1 = scat_e2(e2, dst1, z32, src1)
    h2p = node1(h1mp, pk4(h0p), as3p(agg1))
    h2 = h2p.reshape(npad, 32)
    s1 = smooth1(h2, dst1, z32, src1)
    out = final(as3p(s1), rcp4, pk4(h0p))
    return out.reshape(npad, 8)[:n, :3]


# speedup vs baseline: 1.0171x; 1.0171x over previous
import functools

import jax
import jax.numpy as jnp
from jax import lax
from jax.experimental import pallas as pl
from jax.experimental.pallas import tpu as pltpu
from jax.experimental.pallas import tpu_sc as plsc

NC = 2
NS = 16
NW = NC * NS
CHUNK = 128


def _mesh():
    return plsc.VectorSubcoreMesh(core_axis_name="c", subcore_axis_name="s")


_SC_PARAMS = pltpu.CompilerParams(use_tc_tiling_on_sc=False)


def _sc_gather2(npad, d, epad, gsz, count=False, name="sc_gather"):
    per_w = epad // NW
    rr = 40 * CHUNK
    nb0 = (2 * per_w // rr) * 6 // 10
    nb1 = 2 * per_w // rr - nb0
    pw0, pw1 = nb0 * rr, nb1 * rr
    ops = rr // gsz
    rpt = npad // NS

    out_t = [jax.ShapeDtypeStruct((epad, d), jnp.float32),
             jax.ShapeDtypeStruct((epad, d), jnp.float32)]
    scratch = [
        pltpu.VMEM((rr,), jnp.int32),
        pltpu.VMEM((rr,), jnp.int32),
        pltpu.VMEM((2, gsz, d), jnp.float32),
        pltpu.VMEM((2, gsz, d), jnp.float32),
        [pltpu.SemaphoreType.DMA] * 2,
        [pltpu.SemaphoreType.DMA] * 2,
    ]
    if count:
        out_t.append(jax.ShapeDtypeStruct((NC * npad, 16), jnp.float32))
        scratch += [
            pltpu.VMEM((gsz, 16), jnp.float32),
            pltpu.VMEM_SHARED((npad, 16), jnp.float32),
            pltpu.SemaphoreType.DMA,
        ]

    @functools.partial(pl.kernel, out_type=tuple(out_t), mesh=_mesh(),
                       compiler_params=_SC_PARAMS, name=name,
                       scratch_types=scratch)
    def k(table, src1, dst1, zeros16, ones, *refs):
        if count:
            (hs, hd, cnt, src_v, dst_v, rs_v, rd_v, gsem, wsem,
             ones_v, acc, csem) = refs
        else:
            hs, hd, src_v, dst_v, rs_v, rd_v, gsem, wsem = refs
        c = lax.axis_index("c")
        s = lax.axis_index("s")
        base = c * NS * pw0 + s * (pw0 + c * (pw1 - pw0))
        nbc = nb0 + c * (nb1 - nb0)
        if count:
            pltpu.sync_copy(ones, ones_v)
            pltpu.sync_copy(zeros16.at[pl.ds(s * rpt, rpt)],
                            acc.at[pl.ds(s * rpt, rpt)])
            plsc.subcore_barrier()

        def fire_gather(o, slot):
            pltpu.async_copy(table.at[src_v.at[pl.ds(o * gsz, gsz)]],
                             rs_v.at[slot], gsem[slot])
            pltpu.async_copy(table.at[dst_v.at[pl.ds(o * gsz, gsz)]],
                             rd_v.at[slot], gsem[slot])

        def drain(ref, buf, sem):
            pltpu.make_async_copy(ref.at[pl.ds(0, gsz)], buf, sem).wait()

        def outer(ob, carry):
            pltpu.sync_copy(dst1.at[pl.ds(base + ob * rr, rr)], dst_v)
            pltpu.sync_copy(src1.at[pl.ds(base + ob * rr, rr)], src_v)
            fire_gather(0, 0)

            def body(ip, carry2):
                for b in range(2):
                    o = ip * 2 + b
                    row = base + ob * rr + o * gsz
                    drain(hs, rs_v.at[b], gsem[b])
                    drain(hd, rd_v.at[b], gsem[b])
                    pltpu.async_copy(rs_v.at[b], hs.at[pl.ds(row, gsz)],
                                     wsem[b])
                    pltpu.async_copy(rd_v.at[b], hd.at[pl.ds(row, gsz)],
                                     wsem[b])
                    if count:
                        pltpu.async_copy(
                            ones_v, acc.at[dst_v.at[pl.ds(o * gsz, gsz)]],
                            csem, add=True)

                        @pl.when(ob * ops + o >= 2)
                        def _():
                            pltpu.make_async_copy(ones, ones_v,
                                                  csem).wait()
                    b1 = (b + 1) % 2

                    @pl.when(o + 1 < ops)
                    def _():
                        @pl.when(o >= 1)
                        def _():
                            drain(hs, rs_v.at[b1], wsem[b1])
                            drain(hd, rd_v.at[b1], wsem[b1])
                        fire_gather(o + 1, b1)
                return carry2

            lax.fori_loop(0, ops // 2, body, None)
            for b in range(2):
                drain(hs, rs_v.at[b], wsem[b])
                drain(hd, rd_v.at[b], wsem[b])
            return carry

        lax.fori_loop(0, nbc, outer, None)
        if count:
            pltpu.make_async_copy(ones, ones_v, csem).wait()
            pltpu.make_async_copy(ones, ones_v, csem).wait()
            plsc.subcore_barrier()
            pltpu.sync_copy(acc.at[pl.ds(s * rpt, rpt)],
                            cnt.at[pl.ds(c * npad + s * rpt, rpt)])

    return k


def _sc_scatter_add(npad, width, epad, gather_table=False,
                    name="sc_scatter"):
    per_w = epad // NW
    gsz = 2 * CHUNK
    rr = 20 * CHUNK
    nb0 = (2 * per_w // rr) * 6 // 10
    nb1 = 2 * per_w // rr - nb0
    pw0, pw1 = nb0 * rr, nb1 * rr
    ops = rr // gsz
    rpt = npad // NS

    out_t = jax.ShapeDtypeStruct((NC * npad, width), jnp.float32)
    scratch = [
        pltpu.VMEM((rr,), jnp.int32),
        pltpu.VMEM((rr,), jnp.int32),
        pltpu.VMEM((2, gsz, width), jnp.float32),
        pltpu.VMEM_SHARED((npad, width), jnp.float32),
        [pltpu.SemaphoreType.DMA] * 2,
        [pltpu.SemaphoreType.DMA] * 2,
    ]

    @functools.partial(pl.kernel, out_type=out_t, mesh=_mesh(),
                       compiler_params=_SC_PARAMS, name=name,
                       scratch_types=scratch)
    def k(src_data, dst1, zeros, src1, out, dst_v, src_v, buf, acc,
          lsem, ssem):
        c = lax.axis_index("c")
        s = lax.axis_index("s")
        pltpu.sync_copy(zeros.at[pl.ds(s * rpt, rpt)],
                        acc.at[pl.ds(s * rpt, rpt)])
        plsc.subcore_barrier()
        base = c * NS * pw0 + s * (pw0 + c * (pw1 - pw0))
        nbc = nb0 + c * (nb1 - nb0)

        def start(o, ob, b):
            if gather_table:
                pltpu.async_copy(
                    src_data.at[src_v.at[pl.ds(o * gsz, gsz)]],
                    buf.at[b], lsem[b])
            else:
                pltpu.async_copy(
                    src_data.at[pl.ds(base + ob * rr + o * gsz, gsz)],
                    buf.at[b], lsem[b])

        def outer(ob, carry):
            pltpu.sync_copy(dst1.at[pl.ds(base + ob * rr, rr)], dst_v)
            if gather_table:
                pltpu.sync_copy(src1.at[pl.ds(base + ob * rr, rr)], src_v)
            start(0, ob, 0)

            def body(ip, carry2):
                for b in range(2):
                    o = ip * 2 + b
                    b1 = (b + 1) % 2
                    pltpu.make_async_copy(src_data.at[pl.ds(0, gsz)],
                                          buf.at[b], lsem[b]).wait()
                    pltpu.async_copy(buf.at[b],
                                     acc.at[dst_v.at[pl.ds(o * gsz, gsz)]],
                                     ssem[b], add=True)

                    @pl.when(o + 1 < ops)
                    def _():
                        @pl.when(o >= 1)
                        def _():
                            pltpu.make_async_copy(
                                src_data.at[pl.ds(0, gsz)],
                                buf.at[b1], ssem[b1]).wait()
                        start(o + 1, ob, b1)
                return carry2

            lax.fori_loop(0, ops // 2, body, None)
            for b in range(2):
                pltpu.make_async_copy(src_data.at[pl.ds(0, gsz)],
                                      buf.at[b], ssem[b]).wait()
            return carry

        lax.fori_loop(0, nbc, outer, None)
        plsc.subcore_barrier()
        pltpu.sync_copy(acc.at[pl.ds(s * rpt, rpt)],
                        out.at[pl.ds(c * npad + s * rpt, rpt)])

    return k


def _full(shape):
    return pl.BlockSpec(shape, lambda i: tuple(0 for _ in shape))


def _tc_edge_mlp0(epad, be):
    grid = epad // be
    be4 = be // 4

    def body(hs, hd, ef, a, b, cc, b1, w2, b2, out):
        res = []
        for kk in range(4):
            sl = slice(32 * kk, 32 * kk + 32)
            z = jnp.dot(hs[:, sl], a[...],
                        preferred_element_type=jnp.float32)
            z += jnp.dot(hd[:, sl], b[...],
                         preferred_element_type=jnp.float32)
            z += lax.dot_general(ef[kk], cc[...], (((0,), (0,)), ((), ())),
                                 preferred_element_type=jnp.float32)
            z = jnp.maximum(z + b1[...], 0.0)
            res.append(jnp.dot(z, w2[...],
                               preferred_element_type=jnp.float32)
                       + b2[...])
        out[...] = jnp.concatenate(res, axis=1)

    def make(a, b, cc, b1, w2, b2):
        call = pl.pallas_call(
            body, grid=(grid,), name="tc_edge_mlp0",
            in_specs=[
                pl.BlockSpec((be4, 128), lambda i: (i, 0)),
                pl.BlockSpec((be4, 128), lambda i: (i, 0)),
                pl.BlockSpec((4, 4, be4), lambda i: (0, 0, i)),
                _full(a.shape), _full(b.shape), _full(cc.shape),
                _full(b1.shape), _full(w2.shape), _full(b2.shape),
            ],
            out_specs=pl.BlockSpec((be4, 128), lambda i: (i, 0)),
            out_shape=jax.ShapeDtypeStruct((epad // 4, 128), jnp.float32),
        )
        return lambda hs, hd, ef: call(hs, hd, ef, a, b, cc, b1, w2, b2)

    return make


def _tc_edge_mlp1(epad, be):
    grid = epad // be
    be4 = be // 4

    def body(hs1, hd1, hs0, hd0, ef, ah, ax, bh, bx, cc, b1, w2, b2, out):
        res = []
        for kk in range(4):
            sl = slice(32 * kk, 32 * kk + 32)
            z = jnp.dot(hs1[:, sl], ah[...],
                        preferred_element_type=jnp.float32)
            z += jnp.dot(hs0[:, sl], ax[...],
                         preferred_element_type=jnp.float32)
            z += jnp.dot(hd1[:, sl], bh[...],
                         preferred_element_type=jnp.float32)
            z += jnp.dot(hd0[:, sl], bx[...],
                         preferred_element_type=jnp.float32)
            z += jnp.dot(ef[:, sl], cc[...],
                         preferred_element_type=jnp.float32)
            z = jnp.maximum(z + b1[...], 0.0)
            res.append(jnp.dot(z, w2[...],
                               preferred_element_type=jnp.float32)
                       + b2[...])
        out[...] = jnp.concatenate(res, axis=1)

    def make(ah, ax, bh, bx, cc, b1, w2, b2):
        call = pl.pallas_call(
            body, grid=(grid,), name="tc_edge_mlp1",
            in_specs=[
                pl.BlockSpec((be4, 128), lambda i: (i, 0)),
                pl.BlockSpec((be4, 128), lambda i: (i, 0)),
                pl.BlockSpec((be4, 128), lambda i: (i, 0)),
                pl.BlockSpec((be4, 128), lambda i: (i, 0)),
                pl.BlockSpec((be4, 128), lambda i: (i, 0)),
                _full(ah.shape), _full(ax.shape), _full(bh.shape),
                _full(bx.shape), _full(cc.shape), _full(b1.shape),
                _full(w2.shape), _full(b2.shape),
            ],
            out_specs=pl.BlockSpec((be4, 128), lambda i: (i, 0)),
            out_shape=jax.ShapeDtypeStruct((epad // 4, 128), jnp.float32),
        )
        return lambda hs1, hd1, hs0, hd0, ef: call(
            hs1, hd1, hs0, hd0, ef, ah, ax, bh, bx, cc, b1, w2, b2)

    return make


def _tc_node_mlp(npad, bn, extra=False):
    grid = npad // bn
    bn4 = bn // 4

    def body(*args):
        if extra:
            h, hx, aggp, d1, dx, d2, b1, w2, b2, out = args
        else:
            h, aggp, d1, d2, b1, w2, b2, out = args
        agg = aggp[0] + aggp[1]
        res = []
        for kk in range(4):
            sl = slice(32 * kk, 32 * kk + 32)
            z = jnp.dot(h[:, sl], d1[...],
                        preferred_element_type=jnp.float32)
            if extra:
                z += jnp.dot(hx[:, sl], dx[...],
                             preferred_element_type=jnp.float32)
            z += jnp.dot(agg[:, sl], d2[...],
                         preferred_element_type=jnp.float32)
            z = jnp.maximum(z + b1[...], 0.0)
            res.append(jnp.dot(z, w2[...],
                               preferred_element_type=jnp.float32)
                       + b2[...])
        out[...] = jnp.concatenate(res, axis=1)

    def make(d1, dx, d2, b1, w2, b2):
        specs = [pl.BlockSpec((bn4, 128), lambda i: (i, 0))]
        if extra:
            specs.append(pl.BlockSpec((bn4, 128), lambda i: (i, 0)))
        specs.append(pl.BlockSpec((NC, bn4, 128), lambda i: (0, i, 0)))
        specs.append(_full(d1.shape))
        if extra:
            specs.append(_full(dx.shape))
        specs += [_full(d2.shape), _full(b1.shape), _full(w2.shape),
                  _full(b2.shape)]
        call = pl.pallas_call(
            body, grid=(grid,), name="tc_node_mlp",
            in_specs=specs,
            out_specs=pl.BlockSpec((bn4, 128), lambda i: (i, 0)),
            out_shape=jax.ShapeDtypeStruct((npad // 4, 128), jnp.float32),
        )
        if extra:
            return lambda h, hx, aggp: call(h, hx, aggp, d1, dx, d2, b1,
                                            w2, b2)
        return lambda h, aggp: call(h, aggp, d1, d2, b1, w2, b2)

    return make


def _tc_divide(npad, bn):
    grid = npad // bn
    bn4 = bn // 4

    def body(sp, rcp, out):
        out[...] = (sp[0] + sp[1]) * rcp[...]

    return pl.pallas_call(
        body, grid=(grid,), name="tc_divide",
        in_specs=[
            pl.BlockSpec((NC, bn4, 128), lambda i: (0, i, 0)),
            pl.BlockSpec((bn4, 128), lambda i: (i, 0)),
        ],
        out_specs=pl.BlockSpec((bn4, 128), lambda i: (i, 0)),
        out_shape=jax.ShapeDtypeStruct((npad // 4, 128), jnp.float32),
    )


def _tc_final(npad, bn):
    grid = npad // bn
    bn4 = bn // 4

    def body(sp, rcp, h0, wa, wb, bd, out):
        hm = (sp[0] + sp[1]) * rcp[...]
        res = []
        for kk in range(4):
            z = jnp.dot(hm[:, 32 * kk:32 * kk + 32], wa[...],
                        preferred_element_type=jnp.float32)
            z += jnp.dot(h0[:, 32 * kk:32 * kk + 2], wb[...],
                         preferred_element_type=jnp.float32)
            res.append(z + bd[...])
        out[...] = jnp.concatenate(res, axis=1)

    def make(wa, wb, bd):
        call = pl.pallas_call(
            body, grid=(grid,), name="tc_final",
            in_specs=[
                pl.BlockSpec((NC, bn4, 128), lambda i: (0, i, 0)),
                pl.BlockSpec((bn4, 128), lambda i: (i, 0)),
                pl.BlockSpec((bn4, 128), lambda i: (i, 0)),
                _full(wa.shape), _full(wb.shape), _full(bd.shape),
            ],
            out_specs=pl.BlockSpec((bn4, 32), lambda i: (i, 0)),
            out_shape=jax.ShapeDtypeStruct((npad // 4, 32), jnp.float32),
        )
        return lambda sp, rcp, h0: call(sp, rcp, h0, wa, wb, bd)

    return make


def _pad_rows(w, rows):
    return jnp.concatenate(
        [w, jnp.zeros((rows - w.shape[0], w.shape[1]), w.dtype)], axis=0)


def kernel(x, edge_index, edge_attr, params):
    n = x.shape[0]
    e = edge_index.shape[1]
    npad = ((n + 16) + 127) // 128 * 128
    epad = -(-e // (NW * CHUNK * 8)) * (NW * CHUNK * 8)
    bn = npad // 4
    be = 4096

    src = edge_index[0].astype(jnp.int32)
    dst = edge_index[1].astype(jnp.int32)
    src1 = jnp.concatenate([src, jnp.zeros((epad - e,), jnp.int32)])
    dst1 = jnp.concatenate([dst, jnp.full((epad - e,), n, jnp.int32)])
    eat = jnp.concatenate(
        [edge_attr.T.astype(jnp.float32),
         jnp.zeros((4, epad - e), jnp.float32)], axis=1)
    eatp = jnp.stack(
        [lax.slice(eat, (0, k), (4, epad), (1, 4)) for k in range(4)])

    z32 = jnp.zeros((npad, 32), jnp.float32)
    z16 = jnp.zeros((npad, 16), jnp.float32)
    ones16 = jnp.ones((2 * CHUNK, 16), jnp.float32)

    h0p = _pad_rows(jnp.concatenate(
        [x.astype(jnp.float32), jnp.zeros((n, 26), jnp.float32)], axis=1),
        npad)

    p0, p1 = params["proc0"], params["proc1"]
    row = lambda v: v.reshape(1, -1).astype(jnp.float32)
    f32 = lambda v: v.astype(jnp.float32)

    def xmap(wrows):
        m = jnp.zeros((32, 64), jnp.float32)
        m = m.at[0:2].set(wrows[0:2])
        m = m.at[3:6].set(wrows[2:5])
        return m

    a0 = _pad_rows(f32(p0["We1"][0:6]), 32)
    b0 = _pad_rows(f32(p0["We1"][6:12]), 32)
    c0 = f32(p0["We1"][12:16])
    d1_0 = _pad_rows(f32(p0["Wn1"][0:6]), 32)
    d2_0 = f32(p0["Wn1"][6:38])
    a1h, a1x = f32(p1["We1"][0:32]), xmap(f32(p1["We1"][32:37]))
    b1h, b1x = f32(p1["We1"][37:69]), xmap(f32(p1["We1"][69:74]))
    c1 = f32(p1["We1"][74:106])
    d1h, d1x = f32(p1["Wn1"][0:32]), xmap(f32(p1["Wn1"][32:37]))
    d2_1 = f32(p1["Wn1"][37:69])
    wda = jnp.concatenate(
        [f32(params["Wd"][0:32]), jnp.zeros((32, 5), jnp.float32)], axis=1)
    wdb = jnp.concatenate(
        [f32(params["Wd"][32:34]), jnp.zeros((2, 5), jnp.float32)], axis=1)
    bdp = jnp.concatenate(
        [row(params["bd"]), jnp.zeros((1, 5), jnp.float32)], axis=1)

    gather_l0 = _sc_gather2(npad, 32, epad, 2 * CHUNK, count=True,
                            name="sc_gather_l0")
    gather_l1 = _sc_gather2(npad, 32, epad, 4 * CHUNK, name="sc_gather_l1")
    scat_e1 = _sc_scatter_add(npad, 32, epad, name="sc_seg_e1")
    scat_e2 = _sc_scatter_add(npad, 32, epad, name="sc_seg_e2")
    smooth0 = _sc_scatter_add(npad, 32, epad, gather_table=True,
                              name="sc_smooth0")
    smooth1 = _sc_scatter_add(npad, 32, epad, gather_table=True,
                              name="sc_smooth1")
    edge0 = _tc_edge_mlp0(epad, be)(
        a0, b0, c0, row(p0["be1"]), f32(p0["We2"]), row(p0["be2"]))
    edge1 = _tc_edge_mlp1(epad, be)(
        a1h, a1x, b1h, b1x, c1, row(p1["be1"]), f32(p1["We2"]),
        row(p1["be2"]))
    node0 = _tc_node_mlp(npad, bn)(
        d1_0, None, d2_0, row(p0["bn1"]), f32(p0["Wn2"]), row(p0["bn2"]))
    node1 = _tc_node_mlp(npad, bn, extra=True)(
        d1h, d1x, d2_1, row(p1["bn1"]), f32(p1["Wn2"]), row(p1["bn2"]))
    divide = _tc_divide(npad, bn)
    final = _tc_final(npad, bn)(wda, wdb, bdp)

    pk4 = lambda v: v.reshape(v.shape[0] // 4, 128)
    as3p = lambda v: v.reshape(NC, npad // 4, 128)

    hs0, hd0, cnt = gather_l0(h0p, src1, dst1, z16, ones16)
    e1p = edge0(pk4(hs0), pk4(hd0), eatp)
    e1 = e1p.reshape(epad, 32)
    agg0 = scat_e1(e1, dst1, z32, src1)
    h1p = node0(pk4(h0p), as3p(agg0))
    h1 = h1p.reshape(npad, 32)
    s0 = smooth0(h1, dst1, z32, src1)
    cnt2 = cnt.reshape(NC, npad, 16)
    rcp = 1.0 / jnp.maximum(cnt2[0, :, 0:1] + cnt2[1, :, 0:1], 1.0)
    rcp4 = pk4(jnp.broadcast_to(rcp, (npad, 32)).reshape(npad, 32))
    h1mp = divide(as3p(s0), rcp4)
    h1m = h1mp.reshape(npad, 32)
    hs1, hd1 = gather_l1(h1m, src1, dst1, z16, ones16)
    e2p = edge1(pk4(hs1), pk4(hd1), pk4(hs0), pk4(hd0), e1p)
    e2 = e2p.reshape(epad, 32)
    agg1 = scat_e2(e2, dst1, z32, src1)
    h2p = node1(h1mp, pk4(h0p), as3p(agg1))
    h2 = h2p.reshape(npad, 32)
    s1 = smooth1(h2, dst1, z32, src1)
    out = final(as3p(s1), rcp4, pk4(h0p))
    return out.reshape(npad, 8)[:n, :3]


# speedup vs baseline: 1.0583x; 1.0405x over previous
import functools

import jax
import jax.numpy as jnp
from jax import lax
from jax.experimental import pallas as pl
from jax.experimental.pallas import tpu as pltpu
from jax.experimental.pallas import tpu_sc as plsc

NC = 2
NS = 16
NW = NC * NS
CHUNK = 128


def _mesh():
    return plsc.VectorSubcoreMesh(core_axis_name="c", subcore_axis_name="s")


_SC_PARAMS = pltpu.CompilerParams(use_tc_tiling_on_sc=False)


def _sc_gather2(npad, d, epad, gsz, count=False, name="sc_gather"):
    per_w = epad // NW
    rr = 40 * CHUNK
    nb0 = (2 * per_w // rr) * 6 // 10
    nb1 = 2 * per_w // rr - nb0
    pw0, pw1 = nb0 * rr, nb1 * rr
    ops = rr // gsz
    rpt = npad // NS

    out_t = [jax.ShapeDtypeStruct((epad, d), jnp.float32),
             jax.ShapeDtypeStruct((epad, d), jnp.float32)]
    scratch = [
        pltpu.VMEM((rr,), jnp.int32),
        pltpu.VMEM((rr,), jnp.int32),
        pltpu.VMEM((2, gsz, d), jnp.float32),
        pltpu.VMEM((2, gsz, d), jnp.float32),
        [pltpu.SemaphoreType.DMA] * 2,
        [pltpu.SemaphoreType.DMA] * 2,
    ]
    if count:
        out_t.append(jax.ShapeDtypeStruct((NC * npad, 16), jnp.float32))
        scratch += [
            pltpu.VMEM((gsz, 16), jnp.float32),
            pltpu.VMEM_SHARED((npad, 16), jnp.float32),
            pltpu.SemaphoreType.DMA,
        ]

    @functools.partial(pl.kernel, out_type=tuple(out_t), mesh=_mesh(),
                       compiler_params=_SC_PARAMS, name=name,
                       scratch_types=scratch)
    def k(table, src1, dst1, zeros16, ones, *refs):
        if count:
            (hs, hd, cnt, src_v, dst_v, rs_v, rd_v, gsem, wsem,
             ones_v, acc, csem) = refs
        else:
            hs, hd, src_v, dst_v, rs_v, rd_v, gsem, wsem = refs
        c = lax.axis_index("c")
        s = lax.axis_index("s")
        base = c * NS * pw0 + s * (pw0 + c * (pw1 - pw0))
        nbc = nb0 + c * (nb1 - nb0)
        if count:
            pltpu.sync_copy(ones, ones_v)
            pltpu.sync_copy(zeros16.at[pl.ds(s * rpt, rpt)],
                            acc.at[pl.ds(s * rpt, rpt)])
            plsc.subcore_barrier()

        def fire_gather(o, slot):
            pltpu.async_copy(table.at[src_v.at[pl.ds(o * gsz, gsz)]],
                             rs_v.at[slot], gsem[slot])
            pltpu.async_copy(table.at[dst_v.at[pl.ds(o * gsz, gsz)]],
                             rd_v.at[slot], gsem[slot])

        def drain(ref, buf, sem):
            pltpu.make_async_copy(ref.at[pl.ds(0, gsz)], buf, sem).wait()

        def outer(ob, carry):
            pltpu.sync_copy(dst1.at[pl.ds(base + ob * rr, rr)], dst_v)
            pltpu.sync_copy(src1.at[pl.ds(base + ob * rr, rr)], src_v)
            fire_gather(0, 0)

            def body(ip, carry2):
                for b in range(2):
                    o = ip * 2 + b
                    row = base + ob * rr + o * gsz
                    drain(hs, rs_v.at[b], gsem[b])
                    drain(hd, rd_v.at[b], gsem[b])
                    pltpu.async_copy(rs_v.at[b], hs.at[pl.ds(row, gsz)],
                                     wsem[b])
                    pltpu.async_copy(rd_v.at[b], hd.at[pl.ds(row, gsz)],
                                     wsem[b])
                    if count:
                        pltpu.async_copy(
                            ones_v, acc.at[dst_v.at[pl.ds(o * gsz, gsz)]],
                            csem, add=True)

                        @pl.when(ob * ops + o >= 2)
                        def _():
                            pltpu.make_async_copy(ones, ones_v,
                                                  csem).wait()
                    b1 = (b + 1) % 2

                    @pl.when(o + 1 < ops)
                    def _():
                        @pl.when(o >= 1)
                        def _():
                            drain(hs, rs_v.at[b1], wsem[b1])
                            drain(hd, rd_v.at[b1], wsem[b1])
                        fire_gather(o + 1, b1)
                return carry2

            lax.fori_loop(0, ops // 2, body, None)
            for b in range(2):
                drain(hs, rs_v.at[b], wsem[b])
                drain(hd, rd_v.at[b], wsem[b])
            return carry

        lax.fori_loop(0, nbc, outer, None)
        if count:
            pltpu.make_async_copy(ones, ones_v, csem).wait()
            pltpu.make_async_copy(ones, ones_v, csem).wait()
            plsc.subcore_barrier()
            pltpu.sync_copy(acc.at[pl.ds(s * rpt, rpt)],
                            cnt.at[pl.ds(c * npad + s * rpt, rpt)])

    return k


def _sc_scatter_add(npad, width, epad, gather_table=False,
                    name="sc_scatter"):
    per_w = epad // NW
    gsz = 2 * CHUNK
    rr = 20 * CHUNK
    nb0 = (2 * per_w // rr) * 6 // 10
    nb1 = 2 * per_w // rr - nb0
    pw0, pw1 = nb0 * rr, nb1 * rr
    ops = rr // gsz
    rpt = npad // NS

    out_t = jax.ShapeDtypeStruct((NC * npad, width), jnp.float32)
    scratch = [
        pltpu.VMEM((rr,), jnp.int32),
        pltpu.VMEM((rr,), jnp.int32),
        pltpu.VMEM((2, gsz, width), jnp.float32),
        pltpu.VMEM_SHARED((npad, width), jnp.float32),
        [pltpu.SemaphoreType.DMA] * 2,
        [pltpu.SemaphoreType.DMA] * 2,
    ]

    @functools.partial(pl.kernel, out_type=out_t, mesh=_mesh(),
                       compiler_params=_SC_PARAMS, name=name,
                       scratch_types=scratch)
    def k(src_data, dst1, zeros, src1, out, dst_v, src_v, buf, acc,
          lsem, ssem):
        c = lax.axis_index("c")
        s = lax.axis_index("s")
        pltpu.sync_copy(zeros.at[pl.ds(s * rpt, rpt)],
                        acc.at[pl.ds(s * rpt, rpt)])
        plsc.subcore_barrier()
        base = c * NS * pw0 + s * (pw0 + c * (pw1 - pw0))
        nbc = nb0 + c * (nb1 - nb0)

        def start(o, ob, b):
            if gather_table:
                pltpu.async_copy(
                    src_data.at[src_v.at[pl.ds(o * gsz, gsz)]],
                    buf.at[b], lsem[b])
            else:
                pltpu.async_copy(
                    src_data.at[pl.ds(base + ob * rr + o * gsz, gsz)],
                    buf.at[b], lsem[b])

        def outer(ob, carry):
            pltpu.sync_copy(dst1.at[pl.ds(base + ob * rr, rr)], dst_v)
            if gather_table:
                pltpu.sync_copy(src1.at[pl.ds(base + ob * rr, rr)], src_v)
            start(0, ob, 0)

            def body(ip, carry2):
                for b in range(2):
                    o = ip * 2 + b
                    b1 = (b + 1) % 2
                    pltpu.make_async_copy(src_data.at[pl.ds(0, gsz)],
                                          buf.at[b], lsem[b]).wait()
                    pltpu.async_copy(buf.at[b],
                                     acc.at[dst_v.at[pl.ds(o * gsz, gsz)]],
                                     ssem[b], add=True)

                    @pl.when(o + 1 < ops)
                    def _():
                        @pl.when(o >= 1)
                        def _():
                            pltpu.make_async_copy(
                                src_data.at[pl.ds(0, gsz)],
                                buf.at[b1], ssem[b1]).wait()
                        start(o + 1, ob, b1)
                return carry2

            lax.fori_loop(0, ops // 2, body, None)
            for b in range(2):
                pltpu.make_async_copy(src_data.at[pl.ds(0, gsz)],
                                      buf.at[b], ssem[b]).wait()
            return carry

        lax.fori_loop(0, nbc, outer, None)
        plsc.subcore_barrier()
        pltpu.sync_copy(acc.at[pl.ds(s * rpt, rpt)],
                        out.at[pl.ds(c * npad + s * rpt, rpt)])

    return k


def _full(shape):
    return pl.BlockSpec(shape, lambda i: tuple(0 for _ in shape))


def _tc_edge_mlp0(epad, be):
    grid = epad // be
    be4 = be // 4

    def body(hs, hd, ef, a, b, cc, b1, w2, b2, out):
        res = []
        for kk in range(4):
            sl = slice(32 * kk, 32 * kk + 32)
            z = jnp.dot(hs[:, sl], a[...],
                        preferred_element_type=jnp.float32)
            z += jnp.dot(hd[:, sl], b[...],
                         preferred_element_type=jnp.float32)
            z += lax.dot_general(ef[kk], cc[...], (((0,), (0,)), ((), ())),
                                 preferred_element_type=jnp.float32)
            z = jnp.maximum(z + b1[...], 0.0)
            res.append(jnp.dot(z, w2[...],
                               preferred_element_type=jnp.float32)
                       + b2[...])
        out[...] = jnp.concatenate(res, axis=1)

    def make(a, b, cc, b1, w2, b2):
        call = pl.pallas_call(
            body, grid=(grid,), name="tc_edge_mlp0",
            in_specs=[
                pl.BlockSpec((be4, 128), lambda i: (i, 0)),
                pl.BlockSpec((be4, 128), lambda i: (i, 0)),
                pl.BlockSpec((4, 4, be4), lambda i: (0, 0, i)),
                _full(a.shape), _full(b.shape), _full(cc.shape),
                _full(b1.shape), _full(w2.shape), _full(b2.shape),
            ],
            out_specs=pl.BlockSpec((be4, 128), lambda i: (i, 0)),
            out_shape=jax.ShapeDtypeStruct((epad // 4, 128), jnp.float32),
        )
        return lambda hs, hd, ef: call(hs, hd, ef, a, b, cc, b1, w2, b2)

    return make


def _tc_edge_mlp1(epad, be):
    grid = epad // be
    be4 = be // 4

    def body(hs1, hd1, hs0, hd0, ef, ah, ax, bh, bx, cc, b1, w2, b2, out):
        res = []
        for kk in range(4):
            sl = slice(32 * kk, 32 * kk + 32)
            z = jnp.dot(hs1[:, sl], ah[...],
                        preferred_element_type=jnp.float32)
            z += jnp.dot(hs0[:, sl], ax[...],
                         preferred_element_type=jnp.float32)
            z += jnp.dot(hd1[:, sl], bh[...],
                         preferred_element_type=jnp.float32)
            z += jnp.dot(hd0[:, sl], bx[...],
                         preferred_element_type=jnp.float32)
            z += jnp.dot(ef[:, sl], cc[...],
                         preferred_element_type=jnp.float32)
            z = jnp.maximum(z + b1[...], 0.0)
            res.append(jnp.dot(z, w2[...],
                               preferred_element_type=jnp.float32)
                       + b2[...])
        out[...] = jnp.concatenate(res, axis=1)

    def make(ah, ax, bh, bx, cc, b1, w2, b2):
        call = pl.pallas_call(
            body, grid=(grid,), name="tc_edge_mlp1",
            in_specs=[
                pl.BlockSpec((be4, 128), lambda i: (i, 0)),
                pl.BlockSpec((be4, 128), lambda i: (i, 0)),
                pl.BlockSpec((be4, 128), lambda i: (i, 0)),
                pl.BlockSpec((be4, 128), lambda i: (i, 0)),
                pl.BlockSpec((be4, 128), lambda i: (i, 0)),
                _full(ah.shape), _full(ax.shape), _full(bh.shape),
                _full(bx.shape), _full(cc.shape), _full(b1.shape),
                _full(w2.shape), _full(b2.shape),
            ],
            out_specs=pl.BlockSpec((be4, 128), lambda i: (i, 0)),
            out_shape=jax.ShapeDtypeStruct((epad // 4, 128), jnp.float32),
        )
        return lambda hs1, hd1, hs0, hd0, ef: call(
            hs1, hd1, hs0, hd0, ef, ah, ax, bh, bx, cc, b1, w2, b2)

    return make


def _tc_node_mlp(npad, bn, extra=False):
    grid = npad // bn
    bn4 = bn // 4

    def body(*args):
        if extra:
            h, hx, aggp, d1, dx, d2, b1, w2, b2, out = args
        else:
            h, aggp, d1, d2, b1, w2, b2, out = args
        agg = aggp[0] + aggp[1]
        res = []
        for kk in range(4):
            sl = slice(32 * kk, 32 * kk + 32)
            z = jnp.dot(h[:, sl], d1[...],
                        preferred_element_type=jnp.float32)
            if extra:
                z += jnp.dot(hx[:, sl], dx[...],
                             preferred_element_type=jnp.float32)
            z += jnp.dot(agg[:, sl], d2[...],
                         preferred_element_type=jnp.float32)
            z = jnp.maximum(z + b1[...], 0.0)
            res.append(jnp.dot(z, w2[...],
                               preferred_element_type=jnp.float32)
                       + b2[...])
        out[...] = jnp.concatenate(res, axis=1)

    def make(d1, dx, d2, b1, w2, b2):
        specs = [pl.BlockSpec((bn4, 128), lambda i: (i, 0))]
        if extra:
            specs.append(pl.BlockSpec((bn4, 128), lambda i: (i, 0)))
        specs.append(pl.BlockSpec((NC, bn4, 128), lambda i: (0, i, 0)))
        specs.append(_full(d1.shape))
        if extra:
            specs.append(_full(dx.shape))
        specs += [_full(d2.shape), _full(b1.shape), _full(w2.shape),
                  _full(b2.shape)]
        call = pl.pallas_call(
            body, grid=(grid,), name="tc_node_mlp",
            in_specs=specs,
            out_specs=pl.BlockSpec((bn4, 128), lambda i: (i, 0)),
            out_shape=jax.ShapeDtypeStruct((npad // 4, 128), jnp.float32),
        )
        if extra:
            return lambda h, hx, aggp: call(h, hx, aggp, d1, dx, d2, b1,
                                            w2, b2)
        return lambda h, aggp: call(h, aggp, d1, d2, b1, w2, b2)

    return make


def _tc_divide(npad, bn):
    grid = npad // bn
    bn4 = bn // 4

    def body(sp, rcp, out):
        out[...] = (sp[0] + sp[1]) * rcp[...]

    return pl.pallas_call(
        body, grid=(grid,), name="tc_divide",
        in_specs=[
            pl.BlockSpec((NC, bn4, 128), lambda i: (0, i, 0)),
            pl.BlockSpec((bn4, 128), lambda i: (i, 0)),
        ],
        out_specs=pl.BlockSpec((bn4, 128), lambda i: (i, 0)),
        out_shape=jax.ShapeDtypeStruct((npad // 4, 128), jnp.float32),
    )


def _tc_final(npad, bn):
    grid = npad // bn
    bn4 = bn // 4

    def body(sp, rcp, h0, wa, wb, bd, out):
        hm = (sp[0] + sp[1]) * rcp[...]
        res = []
        for kk in range(4):
            z = jnp.dot(hm[:, 32 * kk:32 * kk + 32], wa[...],
                        preferred_element_type=jnp.float32)
            z += jnp.dot(h0[:, 32 * kk:32 * kk + 2], wb[...],
                         preferred_element_type=jnp.float32)
            res.append(z + bd[...])
        out[...] = jnp.concatenate(res, axis=1)

    def make(wa, wb, bd):
        call = pl.pallas_call(
            body, grid=(grid,), name="tc_final",
            in_specs=[
                pl.BlockSpec((NC, bn4, 128), lambda i: (0, i, 0)),
                pl.BlockSpec((bn4, 128), lambda i: (i, 0)),
                pl.BlockSpec((bn4, 128), lambda i: (i, 0)),
                _full(wa.shape), _full(wb.shape), _full(bd.shape),
            ],
            out_specs=pl.BlockSpec((bn4, 32), lambda i: (i, 0)),
            out_shape=jax.ShapeDtypeStruct((npad // 4, 32), jnp.float32),
        )
        return lambda sp, rcp, h0: call(sp, rcp, h0, wa, wb, bd)

    return make


def _pad_rows(w, rows):
    return jnp.concatenate(
        [w, jnp.zeros((rows - w.shape[0], w.shape[1]), w.dtype)], axis=0)


def kernel(x, edge_index, edge_attr, params):
    n = x.shape[0]
    e = edge_index.shape[1]
    npad = ((n + 16) + 127) // 128 * 128
    epad = -(-e // (NW * CHUNK * 8)) * (NW * CHUNK * 8)
    bn = npad // 4
    be = 8192

    src = edge_index[0].astype(jnp.int32)
    dst = edge_index[1].astype(jnp.int32)
    src1 = jnp.concatenate([src, jnp.zeros((epad - e,), jnp.int32)])
    dst1 = jnp.concatenate([dst, jnp.full((epad - e,), n, jnp.int32)])
    eat = jnp.concatenate(
        [edge_attr.T.astype(jnp.float32),
         jnp.zeros((4, epad - e), jnp.float32)], axis=1)
    eatp = jnp.stack(
        [lax.slice(eat, (0, k), (4, epad), (1, 4)) for k in range(4)])

    z32 = jnp.zeros((npad, 32), jnp.float32)
    z16 = jnp.zeros((npad, 16), jnp.float32)
    ones16 = jnp.ones((2 * CHUNK, 16), jnp.float32)

    h0p = _pad_rows(jnp.concatenate(
        [x.astype(jnp.float32), jnp.zeros((n, 26), jnp.float32)], axis=1),
        npad)

    p0, p1 = params["proc0"], params["proc1"]
    row = lambda v: v.reshape(1, -1).astype(jnp.float32)
    f32 = lambda v: v.astype(jnp.float32)

    def xmap(wrows):
        m = jnp.zeros((32, 64), jnp.float32)
        m = m.at[0:2].set(wrows[0:2])
        m = m.at[3:6].set(wrows[2:5])
        return m

    a0 = _pad_rows(f32(p0["We1"][0:6]), 32)
    b0 = _pad_rows(f32(p0["We1"][6:12]), 32)
    c0 = f32(p0["We1"][12:16])
    d1_0 = _pad_rows(f32(p0["Wn1"][0:6]), 32)
    d2_0 = f32(p0["Wn1"][6:38])
    a1h, a1x = f32(p1["We1"][0:32]), xmap(f32(p1["We1"][32:37]))
    b1h, b1x = f32(p1["We1"][37:69]), xmap(f32(p1["We1"][69:74]))
    c1 = f32(p1["We1"][74:106])
    d1h, d1x = f32(p1["Wn1"][0:32]), xmap(f32(p1["Wn1"][32:37]))
    d2_1 = f32(p1["Wn1"][37:69])
    wda = jnp.concatenate(
        [f32(params["Wd"][0:32]), jnp.zeros((32, 5), jnp.float32)], axis=1)
    wdb = jnp.concatenate(
        [f32(params["Wd"][32:34]), jnp.zeros((2, 5), jnp.float32)], axis=1)
    bdp = jnp.concatenate(
        [row(params["bd"]), jnp.zeros((1, 5), jnp.float32)], axis=1)

    gather_l0 = _sc_gather2(npad, 32, epad, 2 * CHUNK, count=True,
                            name="sc_gather_l0")
    gather_l1 = _sc_gather2(npad, 32, epad, 4 * CHUNK, name="sc_gather_l1")
    scat_e1 = _sc_scatter_add(npad, 32, epad, name="sc_seg_e1")
    scat_e2 = _sc_scatter_add(npad, 32, epad, name="sc_seg_e2")
    smooth0 = _sc_scatter_add(npad, 32, epad, gather_table=True,
                              name="sc_smooth0")
    smooth1 = _sc_scatter_add(npad, 32, epad, gather_table=True,
                              name="sc_smooth1")
    edge0 = _tc_edge_mlp0(epad, be)(
        a0, b0, c0, row(p0["be1"]), f32(p0["We2"]), row(p0["be2"]))
    edge1 = _tc_edge_mlp1(epad, be)(
        a1h, a1x, b1h, b1x, c1, row(p1["be1"]), f32(p1["We2"]),
        row(p1["be2"]))
    node0 = _tc_node_mlp(npad, bn)(
        d1_0, None, d2_0, row(p0["bn1"]), f32(p0["Wn2"]), row(p0["bn2"]))
    node1 = _tc_node_mlp(npad, bn, extra=True)(
        d1h, d1x, d2_1, row(p1["bn1"]), f32(p1["Wn2"]), row(p1["bn2"]))
    divide = _tc_divide(npad, bn)
    final = _tc_final(npad, bn)(wda, wdb, bdp)

    pk4 = lambda v: v.reshape(v.shape[0] // 4, 128)
    as3p = lambda v: v.reshape(NC, npad // 4, 128)

    hs0, hd0, cnt = gather_l0(h0p, src1, dst1, z16, ones16)
    e1p = edge0(pk4(hs0), pk4(hd0), eatp)
    e1 = e1p.reshape(epad, 32)
    agg0 = scat_e1(e1, dst1, z32, src1)
    h1p = node0(pk4(h0p), as3p(agg0))
    h1 = h1p.reshape(npad, 32)
    s0 = smooth0(h1, dst1, z32, src1)
    cnt2 = cnt.reshape(NC, npad, 16)
    rcp = 1.0 / jnp.maximum(cnt2[0, :, 0:1] + cnt2[1, :, 0:1], 1.0)
    rcp4 = pk4(jnp.broadcast_to(rcp, (npad, 32)).reshape(npad, 32))
    h1mp = divide(as3p(s0), rcp4)
    h1m = h1mp.reshape(npad, 32)
    hs1, hd1 = gather_l1(h1m, src1, dst1, z16, ones16)
    e2p = edge1(pk4(hs1), pk4(hd1), pk4(hs0), pk4(hd0), e1p)
    e2 = e2p.reshape(epad, 32)
    agg1 = scat_e2(e2, dst1, z32, src1)
    h2p = node1(h1mp, pk4(h0p), as3p(agg1))
    h2 = h2p.reshape(npad, 32)
    s1 = smooth1(h2, dst1, z32, src1)
    out = final(as3p(s1), rcp4, pk4(h0p))
    return out.reshape(npad, 8)[:n, :3]


# speedup vs baseline: 1.0769x; 1.0175x over previous
import functools

import jax
import jax.numpy as jnp
from jax import lax
from jax.experimental import pallas as pl
from jax.experimental.pallas import tpu as pltpu
from jax.experimental.pallas import tpu_sc as plsc

NC = 2
NS = 16
NW = NC * NS
CHUNK = 128


def _mesh():
    return plsc.VectorSubcoreMesh(core_axis_name="c", subcore_axis_name="s")


_SC_PARAMS = pltpu.CompilerParams(use_tc_tiling_on_sc=False)


def _sc_gather2(npad, d, epad, gsz, count=False, name="sc_gather"):
    per_w = epad // NW
    rr = 40 * CHUNK
    nb0 = (2 * per_w // rr) * 6 // 10
    nb1 = 2 * per_w // rr - nb0
    pw0, pw1 = nb0 * rr, nb1 * rr
    ops = rr // gsz
    rpt = npad // NS

    out_t = [jax.ShapeDtypeStruct((epad, d), jnp.float32),
             jax.ShapeDtypeStruct((epad, d), jnp.float32)]
    scratch = [
        pltpu.VMEM((rr,), jnp.int32),
        pltpu.VMEM((rr,), jnp.int32),
        pltpu.VMEM((2, gsz, d), jnp.float32),
        pltpu.VMEM((2, gsz, d), jnp.float32),
        [pltpu.SemaphoreType.DMA] * 2,
        [pltpu.SemaphoreType.DMA] * 2,
    ]
    if count:
        out_t.append(jax.ShapeDtypeStruct((NC * npad, 16), jnp.float32))
        scratch += [
            pltpu.VMEM((gsz, 16), jnp.float32),
            pltpu.VMEM_SHARED((npad, 16), jnp.float32),
            pltpu.SemaphoreType.DMA,
        ]

    @functools.partial(pl.kernel, out_type=tuple(out_t), mesh=_mesh(),
                       compiler_params=_SC_PARAMS, name=name,
                       scratch_types=scratch)
    def k(table, src1, dst1, zeros16, ones, *refs):
        if count:
            (hs, hd, cnt, src_v, dst_v, rs_v, rd_v, gsem, wsem,
             ones_v, acc, csem) = refs
        else:
            hs, hd, src_v, dst_v, rs_v, rd_v, gsem, wsem = refs
        c = lax.axis_index("c")
        s = lax.axis_index("s")
        base = c * NS * pw0 + s * (pw0 + c * (pw1 - pw0))
        nbc = nb0 + c * (nb1 - nb0)
        if count:
            pltpu.sync_copy(ones, ones_v)
            pltpu.sync_copy(zeros16.at[pl.ds(s * rpt, rpt)],
                            acc.at[pl.ds(s * rpt, rpt)])
            plsc.subcore_barrier()

        def fire_gather(o, slot):
            pltpu.async_copy(table.at[src_v.at[pl.ds(o * gsz, gsz)]],
                             rs_v.at[slot], gsem[slot])
            pltpu.async_copy(table.at[dst_v.at[pl.ds(o * gsz, gsz)]],
                             rd_v.at[slot], gsem[slot])

        def drain(ref, buf, sem):
            pltpu.make_async_copy(ref.at[pl.ds(0, gsz)], buf, sem).wait()

        def outer(ob, carry):
            pltpu.sync_copy(dst1.at[pl.ds(base + ob * rr, rr)], dst_v)
            pltpu.sync_copy(src1.at[pl.ds(base + ob * rr, rr)], src_v)
            fire_gather(0, 0)

            def body(ip, carry2):
                for b in range(2):
                    o = ip * 2 + b
                    row = base + ob * rr + o * gsz
                    drain(hs, rs_v.at[b], gsem[b])
                    drain(hd, rd_v.at[b], gsem[b])
                    pltpu.async_copy(rs_v.at[b], hs.at[pl.ds(row, gsz)],
                                     wsem[b])
                    pltpu.async_copy(rd_v.at[b], hd.at[pl.ds(row, gsz)],
                                     wsem[b])
                    if count:
                        pltpu.async_copy(
                            ones_v, acc.at[dst_v.at[pl.ds(o * gsz, gsz)]],
                            csem, add=True)

                        @pl.when(ob * ops + o >= 2)
                        def _():
                            pltpu.make_async_copy(ones, ones_v,
                                                  csem).wait()
                    b1 = (b + 1) % 2

                    @pl.when(o + 1 < ops)
                    def _():
                        @pl.when(o >= 1)
                        def _():
                            drain(hs, rs_v.at[b1], wsem[b1])
                            drain(hd, rd_v.at[b1], wsem[b1])
                        fire_gather(o + 1, b1)
                return carry2

            lax.fori_loop(0, ops // 2, body, None)
            for b in range(2):
                drain(hs, rs_v.at[b], wsem[b])
                drain(hd, rd_v.at[b], wsem[b])
            return carry

        lax.fori_loop(0, nbc, outer, None)
        if count:
            pltpu.make_async_copy(ones, ones_v, csem).wait()
            pltpu.make_async_copy(ones, ones_v, csem).wait()
            plsc.subcore_barrier()
            pltpu.sync_copy(acc.at[pl.ds(s * rpt, rpt)],
                            cnt.at[pl.ds(c * npad + s * rpt, rpt)])

    return k


def _sc_scatter_add(npad, width, epad, gather_table=False,
                    name="sc_scatter"):
    per_w = epad // NW
    gsz = 2 * CHUNK
    rr = 20 * CHUNK
    nb0 = (2 * per_w // rr) * 6 // 10
    nb1 = 2 * per_w // rr - nb0
    pw0, pw1 = nb0 * rr, nb1 * rr
    ops = rr // gsz
    rpt = npad // NS

    out_t = jax.ShapeDtypeStruct((NC * npad, width), jnp.float32)
    scratch = [
        pltpu.VMEM((rr,), jnp.int32),
        pltpu.VMEM((rr,), jnp.int32),
        pltpu.VMEM((2, gsz, width), jnp.float32),
        pltpu.VMEM_SHARED((npad, width), jnp.float32),
        [pltpu.SemaphoreType.DMA] * 2,
        [pltpu.SemaphoreType.DMA] * 2,
    ]

    @functools.partial(pl.kernel, out_type=out_t, mesh=_mesh(),
                       compiler_params=_SC_PARAMS, name=name,
                       scratch_types=scratch)
    def k(src_data, dst1, zeros, src1, out, dst_v, src_v, buf, acc,
          lsem, ssem):
        c = lax.axis_index("c")
        s = lax.axis_index("s")
        pltpu.sync_copy(zeros.at[pl.ds(s * rpt, rpt)],
                        acc.at[pl.ds(s * rpt, rpt)])
        plsc.subcore_barrier()
        base = c * NS * pw0 + s * (pw0 + c * (pw1 - pw0))
        nbc = nb0 + c * (nb1 - nb0)

        def start(o, ob, b):
            if gather_table:
                pltpu.async_copy(
                    src_data.at[src_v.at[pl.ds(o * gsz, gsz)]],
                    buf.at[b], lsem[b])
            else:
                pltpu.async_copy(
                    src_data.at[pl.ds(base + ob * rr + o * gsz, gsz)],
                    buf.at[b], lsem[b])

        def outer(ob, carry):
            pltpu.sync_copy(dst1.at[pl.ds(base + ob * rr, rr)], dst_v)
            if gather_table:
                pltpu.sync_copy(src1.at[pl.ds(base + ob * rr, rr)], src_v)
            start(0, ob, 0)

            def body(ip, carry2):
                for b in range(2):
                    o = ip * 2 + b
                    b1 = (b + 1) % 2
                    pltpu.make_async_copy(src_data.at[pl.ds(0, gsz)],
                                          buf.at[b], lsem[b]).wait()
                    pltpu.async_copy(buf.at[b],
                                     acc.at[dst_v.at[pl.ds(o * gsz, gsz)]],
                                     ssem[b], add=True)

                    @pl.when(o + 1 < ops)
                    def _():
                        @pl.when(o >= 1)
                        def _():
                            pltpu.make_async_copy(
                                src_data.at[pl.ds(0, gsz)],
                                buf.at[b1], ssem[b1]).wait()
                        start(o + 1, ob, b1)
                return carry2

            lax.fori_loop(0, ops // 2, body, None)
            for b in range(2):
                pltpu.make_async_copy(src_data.at[pl.ds(0, gsz)],
                                      buf.at[b], ssem[b]).wait()
            return carry

        lax.fori_loop(0, nbc, outer, None)
        plsc.subcore_barrier()
        pltpu.sync_copy(acc.at[pl.ds(s * rpt, rpt)],
                        out.at[pl.ds(c * npad + s * rpt, rpt)])

    return k


def _full(shape):
    return pl.BlockSpec(shape, lambda i: tuple(0 for _ in shape))


def _tc_edge_mlp0(epad, be):
    grid = epad // be
    be4 = be // 4

    def body(hs, hd, ef, a, b, cc, b1, w2, b2, out):
        res = []
        for kk in range(4):
            sl = slice(32 * kk, 32 * kk + 32)
            z = jnp.dot(hs[:, sl], a[...],
                        preferred_element_type=jnp.float32)
            z += jnp.dot(hd[:, sl], b[...],
                         preferred_element_type=jnp.float32)
            z += lax.dot_general(ef[kk], cc[...], (((0,), (0,)), ((), ())),
                                 preferred_element_type=jnp.float32)
            z = jnp.maximum(z + b1[...], 0.0)
            res.append(jnp.dot(z, w2[...],
                               preferred_element_type=jnp.float32)
                       + b2[...])
        out[...] = jnp.concatenate(res, axis=1)

    def make(a, b, cc, b1, w2, b2):
        call = pl.pallas_call(
            body, grid=(grid,), name="tc_edge_mlp0",
            in_specs=[
                pl.BlockSpec((be4, 128), lambda i: (i, 0)),
                pl.BlockSpec((be4, 128), lambda i: (i, 0)),
                pl.BlockSpec((4, 4, be4), lambda i: (0, 0, i)),
                _full(a.shape), _full(b.shape), _full(cc.shape),
                _full(b1.shape), _full(w2.shape), _full(b2.shape),
            ],
            out_specs=pl.BlockSpec((be4, 128), lambda i: (i, 0)),
            out_shape=jax.ShapeDtypeStruct((epad // 4, 128), jnp.float32),
        )
        return lambda hs, hd, ef: call(hs, hd, ef, a, b, cc, b1, w2, b2)

    return make


def _tc_edge_mlp1(epad, be):
    grid = epad // be
    be4 = be // 4

    def body(hs1, hd1, hs0, hd0, ef, ah, ax, bh, bx, cc, b1, w2, b2, out):
        res = []
        for kk in range(4):
            sl = slice(32 * kk, 32 * kk + 32)
            z = jnp.dot(hs1[:, sl], ah[...],
                        preferred_element_type=jnp.float32)
            z += jnp.dot(hs0[:, sl], ax[...],
                         preferred_element_type=jnp.float32)
            z += jnp.dot(hd1[:, sl], bh[...],
                         preferred_element_type=jnp.float32)
            z += jnp.dot(hd0[:, sl], bx[...],
                         preferred_element_type=jnp.float32)
            z += jnp.dot(ef[:, sl], cc[...],
                         preferred_element_type=jnp.float32)
            z = jnp.maximum(z + b1[...], 0.0)
            res.append(jnp.dot(z, w2[...],
                               preferred_element_type=jnp.float32)
                       + b2[...])
        out[...] = jnp.concatenate(res, axis=1)

    def make(ah, ax, bh, bx, cc, b1, w2, b2):
        call = pl.pallas_call(
            body, grid=(grid,), name="tc_edge_mlp1",
            in_specs=[
                pl.BlockSpec((be4, 128), lambda i: (i, 0)),
                pl.BlockSpec((be4, 128), lambda i: (i, 0)),
                pl.BlockSpec((be4, 128), lambda i: (i, 0)),
                pl.BlockSpec((be4, 128), lambda i: (i, 0)),
                pl.BlockSpec((be4, 128), lambda i: (i, 0)),
                _full(ah.shape), _full(ax.shape), _full(bh.shape),
                _full(bx.shape), _full(cc.shape), _full(b1.shape),
                _full(w2.shape), _full(b2.shape),
            ],
            out_specs=pl.BlockSpec((be4, 128), lambda i: (i, 0)),
            out_shape=jax.ShapeDtypeStruct((epad // 4, 128), jnp.float32),
        )
        return lambda hs1, hd1, hs0, hd0, ef: call(
            hs1, hd1, hs0, hd0, ef, ah, ax, bh, bx, cc, b1, w2, b2)

    return make


def _tc_node_mlp(npad, bn, extra=False):
    grid = npad // bn
    bn4 = bn // 4

    def body(*args):
        if extra:
            h, hx, aggp, d1, dx, d2, b1, w2, b2, out = args
        else:
            h, aggp, d1, d2, b1, w2, b2, out = args
        agg = aggp[0] + aggp[1]
        res = []
        for kk in range(4):
            sl = slice(32 * kk, 32 * kk + 32)
            z = jnp.dot(h[:, sl], d1[...],
                        preferred_element_type=jnp.float32)
            if extra:
                z += jnp.dot(hx[:, sl], dx[...],
                             preferred_element_type=jnp.float32)
            z += jnp.dot(agg[:, sl], d2[...],
                         preferred_element_type=jnp.float32)
            z = jnp.maximum(z + b1[...], 0.0)
            res.append(jnp.dot(z, w2[...],
                               preferred_element_type=jnp.float32)
                       + b2[...])
        out[...] = jnp.concatenate(res, axis=1)

    def make(d1, dx, d2, b1, w2, b2):
        specs = [pl.BlockSpec((bn4, 128), lambda i: (i, 0))]
        if extra:
            specs.append(pl.BlockSpec((bn4, 128), lambda i: (i, 0)))
        specs.append(pl.BlockSpec((NC, bn4, 128), lambda i: (0, i, 0)))
        specs.append(_full(d1.shape))
        if extra:
            specs.append(_full(dx.shape))
        specs += [_full(d2.shape), _full(b1.shape), _full(w2.shape),
                  _full(b2.shape)]
        call = pl.pallas_call(
            body, grid=(grid,), name="tc_node_mlp",
            in_specs=specs,
            out_specs=pl.BlockSpec((bn4, 128), lambda i: (i, 0)),
            out_shape=jax.ShapeDtypeStruct((npad // 4, 128), jnp.float32),
        )
        if extra:
            return lambda h, hx, aggp: call(h, hx, aggp, d1, dx, d2, b1,
                                            w2, b2)
        return lambda h, aggp: call(h, aggp, d1, d2, b1, w2, b2)

    return make


def _tc_divide(npad, bn):
    grid = npad // bn
    bn4 = bn // 4

    def body(sp, rcp, out):
        out[...] = (sp[0] + sp[1]) * rcp[...]

    return pl.pallas_call(
        body, grid=(grid,), name="tc_divide",
        in_specs=[
            pl.BlockSpec((NC, bn4, 128), lambda i: (0, i, 0)),
            pl.BlockSpec((bn4, 128), lambda i: (i, 0)),
        ],
        out_specs=pl.BlockSpec((bn4, 128), lambda i: (i, 0)),
        out_shape=jax.ShapeDtypeStruct((npad // 4, 128), jnp.float32),
    )


def _tc_final(npad, bn):
    grid = npad // bn
    bn4 = bn // 4

    def body(sp, rcp, h0, wa, wb, bd, out):
        hm = (sp[0] + sp[1]) * rcp[...]
        res = []
        for kk in range(4):
            z = jnp.dot(hm[:, 32 * kk:32 * kk + 32], wa[...],
                        preferred_element_type=jnp.float32)
            z += jnp.dot(h0[:, 32 * kk:32 * kk + 2], wb[...],
                         preferred_element_type=jnp.float32)
            res.append(z + bd[...])
        out[...] = jnp.concatenate(res, axis=1)

    def make(wa, wb, bd):
        call = pl.pallas_call(
            body, grid=(grid,), name="tc_final",
            in_specs=[
                pl.BlockSpec((NC, bn4, 128), lambda i: (0, i, 0)),
                pl.BlockSpec((bn4, 128), lambda i: (i, 0)),
                pl.BlockSpec((bn4, 128), lambda i: (i, 0)),
                _full(wa.shape), _full(wb.shape), _full(bd.shape),
            ],
            out_specs=pl.BlockSpec((bn4, 32), lambda i: (i, 0)),
            out_shape=jax.ShapeDtypeStruct((npad // 4, 32), jnp.float32),
        )
        return lambda sp, rcp, h0: call(sp, rcp, h0, wa, wb, bd)

    return make


def _pad_rows(w, rows):
    return jnp.concatenate(
        [w, jnp.zeros((rows - w.shape[0], w.shape[1]), w.dtype)], axis=0)


def kernel(x, edge_index, edge_attr, params):
    n = x.shape[0]
    e = edge_index.shape[1]
    npad = ((n + 16) + 127) // 128 * 128
    epad = -(-e // (NW * CHUNK * 8)) * (NW * CHUNK * 8)
    bn = npad // 4
    be = 16384

    src = edge_index[0].astype(jnp.int32)
    dst = edge_index[1].astype(jnp.int32)
    src1 = jnp.concatenate([src, jnp.zeros((epad - e,), jnp.int32)])
    dst1 = jnp.concatenate([dst, jnp.full((epad - e,), n, jnp.int32)])
    eat = jnp.concatenate(
        [edge_attr.T.astype(jnp.float32),
         jnp.zeros((4, epad - e), jnp.float32)], axis=1)
    eatp = jnp.stack(
        [lax.slice(eat, (0, k), (4, epad), (1, 4)) for k in range(4)])

    z32 = jnp.zeros((npad, 32), jnp.float32)
    z16 = jnp.zeros((npad, 16), jnp.float32)
    ones16 = jnp.ones((2 * CHUNK, 16), jnp.float32)

    h0p = _pad_rows(jnp.concatenate(
        [x.astype(jnp.float32), jnp.zeros((n, 26), jnp.float32)], axis=1),
        npad)

    p0, p1 = params["proc0"], params["proc1"]
    row = lambda v: v.reshape(1, -1).astype(jnp.float32)
    f32 = lambda v: v.astype(jnp.float32)

    def xmap(wrows):
        m = jnp.zeros((32, 64), jnp.float32)
        m = m.at[0:2].set(wrows[0:2])
        m = m.at[3:6].set(wrows[2:5])
        return m

    a0 = _pad_rows(f32(p0["We1"][0:6]), 32)
    b0 = _pad_rows(f32(p0["We1"][6:12]), 32)
    c0 = f32(p0["We1"][12:16])
    d1_0 = _pad_rows(f32(p0["Wn1"][0:6]), 32)
    d2_0 = f32(p0["Wn1"][6:38])
    a1h, a1x = f32(p1["We1"][0:32]), xmap(f32(p1["We1"][32:37]))
    b1h, b1x = f32(p1["We1"][37:69]), xmap(f32(p1["We1"][69:74]))
    c1 = f32(p1["We1"][74:106])
    d1h, d1x = f32(p1["Wn1"][0:32]), xmap(f32(p1["Wn1"][32:37]))
    d2_1 = f32(p1["Wn1"][37:69])
    wda = jnp.concatenate(
        [f32(params["Wd"][0:32]), jnp.zeros((32, 5), jnp.float32)], axis=1)
    wdb = jnp.concatenate(
        [f32(params["Wd"][32:34]), jnp.zeros((2, 5), jnp.float32)], axis=1)
    bdp = jnp.concatenate(
        [row(params["bd"]), jnp.zeros((1, 5), jnp.float32)], axis=1)

    gather_l0 = _sc_gather2(npad, 32, epad, 2 * CHUNK, count=True,
                            name="sc_gather_l0")
    gather_l1 = _sc_gather2(npad, 32, epad, 4 * CHUNK, name="sc_gather_l1")
    scat_e1 = _sc_scatter_add(npad, 32, epad, name="sc_seg_e1")
    scat_e2 = _sc_scatter_add(npad, 32, epad, name="sc_seg_e2")
    smooth0 = _sc_scatter_add(npad, 32, epad, gather_table=True,
                              name="sc_smooth0")
    smooth1 = _sc_scatter_add(npad, 32, epad, gather_table=True,
                              name="sc_smooth1")
    edge0 = _tc_edge_mlp0(epad, be)(
        a0, b0, c0, row(p0["be1"]), f32(p0["We2"]), row(p0["be2"]))
    edge1 = _tc_edge_mlp1(epad, be)(
        a1h, a1x, b1h, b1x, c1, row(p1["be1"]), f32(p1["We2"]),
        row(p1["be2"]))
    node0 = _tc_node_mlp(npad, bn)(
        d1_0, None, d2_0, row(p0["bn1"]), f32(p0["Wn2"]), row(p0["bn2"]))
    node1 = _tc_node_mlp(npad, bn, extra=True)(
        d1h, d1x, d2_1, row(p1["bn1"]), f32(p1["Wn2"]), row(p1["bn2"]))
    divide = _tc_divide(npad, bn)
    final = _tc_final(npad, bn)(wda, wdb, bdp)

    pk4 = lambda v: v.reshape(v.shape[0] // 4, 128)
    as3p = lambda v: v.reshape(NC, npad // 4, 128)

    hs0, hd0, cnt = gather_l0(h0p, src1, dst1, z16, ones16)
    e1p = edge0(pk4(hs0), pk4(hd0), eatp)
    e1 = e1p.reshape(epad, 32)
    agg0 = scat_e1(e1, dst1, z32, src1)
    h1p = node0(pk4(h0p), as3p(agg0))
    h1 = h1p.reshape(npad, 32)
    s0 = smooth0(h1, dst1, z32, src1)
    cnt2 = cnt.reshape(NC, npad, 16)
    rcp = 1.0 / jnp.maximum(cnt2[0, :, 0:1] + cnt2[1, :, 0:1], 1.0)
    rcp4 = pk4(jnp.broadcast_to(rcp, (npad, 32)).reshape(npad, 32))
    h1mp = divide(as3p(s0), rcp4)
    h1m = h1mp.reshape(npad, 32)
    hs1, hd1 = gather_l1(h1m, src1, dst1, z16, ones16)
    e2p = edge1(pk4(hs1), pk4(hd1), pk4(hs0), pk4(hd0), e1p)
    e2 = e2p.reshape(epad, 32)
    agg1 = scat_e2(e2, dst1, z32, src1)
    h2p = node1(h1mp, pk4(h0p), as3p(agg1))
    h2 = h2p.reshape(npad, 32)
    s1 = smooth1(h2, dst1, z32, src1)
    out = final(as3p(s1), rcp4, pk4(h0p))
    return out.reshape(npad, 8)[:n, :3]
